# baseline (speedup 1.0000x reference)
import jax
import jax.numpy as jnp
from jax.experimental import pallas as pl
from jax.experimental.pallas import tpu as pltpu

B, D, K = 4096, 2048, 8
BM = 256


def _encode_block(idx_ref, mask_ref, x_ref, w_ref, b_ref, out_ref):
    idx = idx_ref[0]
    onehot = (jax.lax.broadcasted_iota(jnp.int32, (1, K), 1) == idx)
    sel = jnp.sum(mask_ref[...] * onehot.astype(jnp.float32), axis=1,
                  keepdims=True)
    keep = sel > 0.5
    xb = x_ref[...].astype(jnp.bfloat16)
    wb = w_ref[...].astype(jnp.bfloat16)
    acc = jax.lax.dot_general(
        xb, wb, (((1,), (1,)), ((), ())),
        preferred_element_type=jnp.float32)
    acc = acc + b_ref[...]
    out_ref[...] = jnp.where(keep, acc, 0.0)


def kernel(input_data, selection_mask, W, bvec, modality_idx):
    idx = jnp.atleast_1d(jnp.asarray(modality_idx, dtype=jnp.int32))
    grid_spec = pltpu.PrefetchScalarGridSpec(
        num_scalar_prefetch=1,
        grid=(B // BM,),
        in_specs=[
            pl.BlockSpec((BM, K), lambda i, *_: (i, 0)),
            pl.BlockSpec((BM, D), lambda i, *_: (i, 0)),
            pl.BlockSpec((D, D), lambda i, *_: (0, 0)),
            pl.BlockSpec((1, D), lambda i, *_: (0, 0)),
        ],
        out_specs=pl.BlockSpec((BM, D), lambda i, *_: (i, 0)),
    )
    return pl.pallas_call(
        _encode_block,
        grid_spec=grid_spec,
        out_shape=jax.ShapeDtypeStruct((B, D), jnp.float32),
        compiler_params=pltpu.CompilerParams(
            dimension_semantics=("parallel",)),
    )(idx, selection_mask, input_data, W, bvec.reshape(1, D))


# speedup vs baseline: 1.0397x; 1.0397x over previous
import jax
import jax.numpy as jnp
from jax.experimental import pallas as pl
from jax.experimental.pallas import tpu as pltpu

B, D, K = 4096, 2048, 8
BM = 512


def _encode_block(idx_ref, mask_ref, x_ref, w_ref, b_ref, out_ref):
    idx = idx_ref[0]
    onehot = (jax.lax.broadcasted_iota(jnp.int32, (1, K), 1) == idx)
    sel = jnp.sum(mask_ref[...] * onehot.astype(jnp.float32), axis=1,
                  keepdims=True)
    keep = sel > 0.5
    xb = x_ref[...].astype(jnp.bfloat16)
    wb = w_ref[...].astype(jnp.bfloat16)
    acc = jax.lax.dot_general(
        xb, wb, (((1,), (1,)), ((), ())),
        preferred_element_type=jnp.float32)
    acc = acc + b_ref[...]
    out_ref[...] = jnp.where(keep, acc, 0.0)


def kernel(input_data, selection_mask, W, bvec, modality_idx):
    idx = jnp.atleast_1d(jnp.asarray(modality_idx, dtype=jnp.int32))
    grid_spec = pltpu.PrefetchScalarGridSpec(
        num_scalar_prefetch=1,
        grid=(B // BM,),
        in_specs=[
            pl.BlockSpec((BM, K), lambda i, *_: (i, 0)),
            pl.BlockSpec((BM, D), lambda i, *_: (i, 0)),
            pl.BlockSpec((D, D), lambda i, *_: (0, 0)),
            pl.BlockSpec((1, D), lambda i, *_: (0, 0)),
        ],
        out_specs=pl.BlockSpec((BM, D), lambda i, *_: (i, 0)),
    )
    return pl.pallas_call(
        _encode_block,
        grid_spec=grid_spec,
        out_shape=jax.ShapeDtypeStruct((B, D), jnp.float32),
        compiler_params=pltpu.CompilerParams(
            dimension_semantics=("parallel",)),
    )(idx, selection_mask, input_data, W, bvec.reshape(1, D))
